# baseline (speedup 1.0000x reference)
import functools

import jax
import jax.numpy as jnp
from jax import lax
from jax.experimental import pallas as pl
from jax.experimental.pallas import tpu as pltpu
from jax.experimental.pallas import tpu_sc as plsc

VOCAB = 1000
BATCH = 4096
SEQ = 20

_INFO = plsc.get_sparse_core_info()
_NC = _INFO.num_cores
_NS = _INFO.num_subcores
_NW = _NC * _NS

_D = VOCAB
_DP = 1024
_BPW = BATCH // _NW
_NDT = _DP // 128
_DLAST = _D - 7 * 128


def _make_kernel():
    mesh = plsc.VectorSubcoreMesh(core_axis_name="c", subcore_axis_name="s")

    @functools.partial(
        pl.kernel,
        mesh=mesh,
        out_type=jax.ShapeDtypeStruct((SEQ, _D, BATCH), jnp.float32),
        scratch_types=(
            [pltpu.VMEM((SEQ, _BPW), jnp.int32)]
            + [pltpu.VMEM((_BPW, 128), jnp.float32) for _ in range(4)]
            + [pltpu.SemaphoreType.DMA for _ in range(4)]
        ),
        compiler_params=pltpu.CompilerParams(needs_layout_passes=False),
    )
    def body(x_hbm, table_hbm, out_hbm, idx_v, bin0, bin1, bout0, bout1,
             gsem0, gsem1, ssem0, ssem1):
        bins = (bin0, bin1)
        bouts = (bout0, bout1)
        gsems = (gsem0, gsem1)
        ssems = (ssem0, ssem1)
        wid = lax.axis_index("s") * _NC + lax.axis_index("c")
        b0 = wid * _BPW
        pltpu.sync_copy(x_hbm.at[wid], idx_v)
        lanes = lax.iota(jnp.int32, 16)

        def gather_args(t, dt, p):
            src = table_hbm.at[idx_v.at[t], pl.ds(dt * 128, 128)]
            return src, bins[p], gsems[p]

        def scatter_args(t, dt, p):
            if dt == _NDT - 1:
                src = bouts[p].at[pl.ds(0, _DLAST), :]
                dst = out_hbm.at[t, pl.ds(dt * 128, _DLAST), pl.ds(b0, _BPW)]
            else:
                src = bouts[p]
                dst = out_hbm.at[t, pl.ds(dt * 128, 128), pl.ds(b0, _BPW)]
            return src, dst, ssems[p]

        zerov = jnp.zeros((16,), jnp.int32)
        lanes128 = lanes * 128
        diags = tuple((lanes + k) & 15 for k in range(16))
        diags128 = tuple(d * 128 for d in diags)

        def transpose_unit(p):
            bin_p, bout_p = bins[p], bouts[p]

            def blk(i, carry):
                bi = i // 8
                di = i - bi * 8
                a = lanes128 + (bi * 2048 + di * 16)
                b = lanes + (di * 2048 + bi * 16)
                for k in range(16):
                    v = plsc.load_gather(bin_p, [zerov, a + diags[k]])
                    plsc.store_scatter(bout_p, [zerov, b + diags128[k]], v)
                return carry

            lax.fori_loop(0, (_BPW // 16) * 8, blk, 0)

        s, d, sem = gather_args(0, 0, 0)
        pltpu.async_copy(s, d, sem)

        def per_t(t, carry):
            for dt in range(_NDT):
                p = dt % 2
                s, d, sem = gather_args(t, dt, p)
                pltpu.make_async_copy(s, d, sem).wait()
                if dt < _NDT - 1:
                    s, d, sem = gather_args(t, dt + 1, 1 - p)
                    pltpu.async_copy(s, d, sem)
                if dt >= 2:
                    s, d, sem = scatter_args(t, dt - 2, p)
                    pltpu.make_async_copy(s, d, sem).wait()
                else:

                    @pl.when(t > 0)
                    def _():
                        s, d, sem = scatter_args(t - 1, _NDT - 2 + dt, p)
                        pltpu.make_async_copy(s, d, sem).wait()

                transpose_unit(p)
                s, d, sem = scatter_args(t, dt, p)
                pltpu.async_copy(s, d, sem)

            @pl.when(t < SEQ - 1)
            def _():
                s, d, sem = gather_args(t + 1, 0, 0)
                pltpu.async_copy(s, d, sem)

            return carry

        lax.fori_loop(0, SEQ, per_t, 0)

        s, d, sem = scatter_args(SEQ - 1, _NDT - 2, 0)
        pltpu.make_async_copy(s, d, sem).wait()
        s, d, sem = scatter_args(SEQ - 1, _NDT - 1, 1)
        pltpu.make_async_copy(s, d, sem).wait()

    return body


_kernel_call = _make_kernel()


def kernel(x, embedding):
    idx = x.astype(jnp.int32).reshape(_NW, _BPW, SEQ).transpose(0, 2, 1)
    table = jnp.pad(embedding, ((0, 0), (0, _DP - _D)))
    out_t = _kernel_call(idx, table)
    return jnp.transpose(out_t, (2, 0, 1))


# speedup vs baseline: 1.3919x; 1.3919x over previous
import functools

import jax
import jax.numpy as jnp
from jax import lax
from jax.experimental import pallas as pl
from jax.experimental.pallas import tpu as pltpu
from jax.experimental.pallas import tpu_sc as plsc

VOCAB = 1000
BATCH = 4096
SEQ = 20

_INFO = plsc.get_sparse_core_info()
_NC = _INFO.num_cores
_NS = _INFO.num_subcores
_NW = _NC * _NS

_D = VOCAB
_DP = 1024
_SP = 24
_BPW = BATCH // _NW
_DEPTH = 4
_C = _SP
_G = _BPW
_NGROUP = _G // _DEPTH


def _make_kernel():
    mesh = plsc.VectorSubcoreMesh(core_axis_name="c", subcore_axis_name="s")

    @functools.partial(
        pl.kernel,
        mesh=mesh,
        out_type=jax.ShapeDtypeStruct((BATCH, _SP, _DP), jnp.float32),
        scratch_types=(
            [pltpu.VMEM((_G, _C), jnp.int32)]
            + [pltpu.VMEM((1, _SP, _DP), jnp.float32)
               for _ in range(_DEPTH)]
            + [pltpu.SemaphoreType.DMA for _ in range(2 * _DEPTH)]
        ),
    )
    def body(x_hbm, table_hbm, out_hbm, idx_v, *rest):
        bufs = rest[:_DEPTH]
        gsems = rest[_DEPTH:2 * _DEPTH]
        ssems = rest[2 * _DEPTH:]
        wid = lax.axis_index("s") * _NC + lax.axis_index("c")
        base = wid * _BPW
        pltpu.sync_copy(x_hbm.at[wid], idx_v)

        def fire_gather(g, j):
            pltpu.async_copy(table_hbm.at[idx_v.at[g]], bufs[j].at[0],
                             gsems[j])

        def wait_gather(g, j):
            pltpu.make_async_copy(
                table_hbm.at[idx_v.at[g]], bufs[j].at[0], gsems[j]).wait()

        def _scatter_args(g, j):
            src = bufs[j]
            dst = out_hbm.at[pl.ds(base + g, 1)]
            return src, dst

        def fire_scatter(g, j):
            src, dst = _scatter_args(g, j)
            pltpu.async_copy(src, dst, ssems[j])

        def wait_scatter(g, j):
            src, dst = _scatter_args(g, j)
            pltpu.make_async_copy(src, dst, ssems[j]).wait()

        for j in range(_DEPTH):
            fire_gather(j, j)

        def group(gg, carry):
            for j in range(_DEPTH):
                g = gg * _DEPTH + j
                wait_gather(g, j)
                fire_scatter(g, j)
            for j in range(_DEPTH):
                g = gg * _DEPTH + j
                wait_scatter(g, j)
                fire_gather(g + _DEPTH, j)
            return carry

        lax.fori_loop(0, _NGROUP - 1, group, 0)

        for j in range(_DEPTH):
            g = (_NGROUP - 1) * _DEPTH + j
            wait_gather(g, j)
            fire_scatter(g, j)
        for j in range(_DEPTH):
            g = (_NGROUP - 1) * _DEPTH + j
            wait_scatter(g, j)

    return body


_kernel_call = _make_kernel()


def kernel(x, embedding):
    xr = x.astype(jnp.int32).reshape(_NW, _BPW, SEQ)
    idx = jnp.concatenate([xr, xr[:, :, : _SP - SEQ]], axis=-1)
    table = jnp.pad(embedding, ((0, 0), (0, _DP - _D)))
    out = _kernel_call(idx, table)
    return out[:, :SEQ, :_D]
